# Initial kernel scaffold; baseline (speedup 1.0000x reference)
import functools

import jax
import jax.numpy as jnp
from jax import lax
from jax.experimental import pallas as pl
from jax.experimental.pallas import tpu as pltpu
from jax.experimental.pallas import tpu_sc as plsc

N_NODES = 10000
N_EDGES = 320000
D = 128
EPS = 0.3

NC = 2
NS = 16
NW = NC * NS
CH = 128
EPT = N_EDGES // NW
NCHUNK = (EPT + CH - 1) // CH
EPT_PAD = NCHUNK * CH
NP = N_NODES + 16
NZ = 10240

_mesh = plsc.VectorSubcoreMesh(core_axis_name="c", subcore_axis_name="s")


@functools.partial(
    pl.kernel,
    out_type=jax.ShapeDtypeStruct((NC, NZ), jnp.float32),
    mesh=_mesh,
    scratch_types=dict(
        deg=pltpu.VMEM_SHARED((NZ,), jnp.float32),
        rowi=pltpu.VMEM((NCHUNK, CH), jnp.int32),
        vbuf=pltpu.VMEM((CH,), jnp.float32),
    ),
)
def _sc_deg(row_hbm, z640_hbm, out_hbm, deg, rowi, vbuf):
    c = lax.axis_index("c")
    s = lax.axis_index("s")
    w = c * NS + s
    pltpu.sync_copy(row_hbm.at[w], rowi)
    pltpu.sync_copy(z640_hbm, deg.at[pl.ds(s * 640, 640)])

    def fill(k, _):
        vbuf[pl.ds(k * 16, 16)] = jnp.full((16,), 1.0, jnp.float32)
        return 0

    lax.fori_loop(0, CH // 16, fill, 0)
    plsc.subcore_barrier()

    def chunk(j, _):
        pltpu.sync_copy(vbuf, deg.at[rowi.at[j]], add=True)
        return 0

    lax.fori_loop(0, NCHUNK, chunk, 0)
    plsc.subcore_barrier()
    pltpu.sync_copy(deg.at[pl.ds(s * 640, 640)], out_hbm.at[c, pl.ds(s * 640, 640)])


@functools.partial(
    pl.kernel,
    out_type=jax.ShapeDtypeStruct((NC, N_NODES, D), jnp.float32),
    mesh=_mesh,
    scratch_types=dict(
        agg=pltpu.VMEM_SHARED((NP, D), jnp.float32),
        rowi=pltpu.VMEM((NCHUNK, CH), jnp.int32),
        coli=pltpu.VMEM((NCHUNK, CH), jnp.int32),
        av=pltpu.VMEM((NP,), jnp.float32),
        bv=pltpu.VMEM((NP,), jnp.float32),
        ndv=pltpu.VMEM((NP,), jnp.float32),
        cbuf=pltpu.VMEM((CH,), jnp.float32),
        rows=pltpu.VMEM((CH, D), jnp.float32),
        sem=pltpu.SemaphoreType.DMA,
    ),
)
def _sc_layer(h_hbm, a_hbm, b_hbm, nd_hbm, row_hbm, col_hbm, zrows_hbm,
              out_hbm, agg, rowi, coli, av, bv, ndv, cbuf, rows, sem):
    c = lax.axis_index("c")
    s = lax.axis_index("s")
    w = c * NS + s
    pltpu.sync_copy(row_hbm.at[w], rowi)
    pltpu.sync_copy(col_hbm.at[w], coli)
    pltpu.sync_copy(a_hbm, av)
    pltpu.sync_copy(b_hbm, bv)
    pltpu.sync_copy(nd_hbm, ndv)
    pltpu.sync_copy(zrows_hbm, agg.at[pl.ds(s * 626, 626)])
    plsc.subcore_barrier()

    def chunk(j, _):
        def sub(k, _):
            ir = rowi[j, pl.ds(k * 16, 16)]
            ic = coli[j, pl.ds(k * 16, 16)]
            ar = plsc.load_gather(av, [ir])
            bc = plsc.load_gather(bv, [ic])
            nr = plsc.load_gather(ndv, [ir])
            nc_ = plsc.load_gather(ndv, [ic])
            t = ar + bc
            e = jnp.exp(-2.0 * jnp.abs(t))
            th = (1.0 - e) / (1.0 + e)
            th = jnp.where(t < 0.0, -th, th)
            cbuf[pl.ds(k * 16, 16)] = th * nr * nc_
            return 0

        lax.fori_loop(0, CH // 16, sub, 0)
        pltpu.async_copy(h_hbm.at[rowi.at[j]], rows, sem).wait()

        def scale(e, _):
            ce = cbuf[e]
            for q in range(D // 16):
                rows[e, pl.ds(q * 16, 16)] = rows[e, pl.ds(q * 16, 16)] * ce
            return 0

        lax.fori_loop(0, CH, scale, 0)
        pltpu.sync_copy(rows, agg.at[coli.at[j]], add=True)
        return 0

    lax.fori_loop(0, NCHUNK, chunk, 0)
    plsc.subcore_barrier()
    npt = N_NODES // NS
    pltpu.sync_copy(agg.at[pl.ds(s * npt, npt)],
                    out_hbm.at[c, pl.ds(s * npt, npt)])


_BN = 1000
_BZ = NZ // 10


def _tc_a_body(x_ref, w1_ref, b1_ref, wg_ref, bg_ref, degp_ref,
               h_ref, a_ref, b_ref, nd_ref):
    h = lax.dot_general(x_ref[...], w1_ref[...], (((1,), (1,)), ((), ())),
                        preferred_element_type=jnp.float32)
    h = jnp.maximum(h + b1_ref[...], 0.0)
    h_ref[...] = h
    g2 = lax.dot_general(wg_ref[...], h, (((1,), (1,)), ((), ())),
                         preferred_element_type=jnp.float32)
    a_ref[...] = g2[0:1, :] + bg_ref[...]
    b_ref[...] = g2[1:2, :]
    deg = degp_ref[0:1, :] + degp_ref[1:2, :]
    nd_ref[...] = lax.rsqrt(jnp.maximum(deg, 1.0))


def _tc_a(x, w1, b1r, wg2, bgr, degp):
    return pl.pallas_call(
        _tc_a_body,
        grid=(N_NODES // _BN,),
        in_specs=[
            pl.BlockSpec((_BN, D), lambda i: (i, 0)),
            pl.BlockSpec((D, D), lambda i: (0, 0)),
            pl.BlockSpec((1, D), lambda i: (0, 0)),
            pl.BlockSpec((2, D), lambda i: (0, 0)),
            pl.BlockSpec((1, 1), lambda i: (0, 0)),
            pl.BlockSpec((2, _BZ), lambda i: (0, i)),
        ],
        out_specs=[
            pl.BlockSpec((_BN, D), lambda i: (i, 0)),
            pl.BlockSpec((1, _BN), lambda i: (0, i)),
            pl.BlockSpec((1, _BN), lambda i: (0, i)),
            pl.BlockSpec((1, _BZ), lambda i: (0, i)),
        ],
        out_shape=[
            jax.ShapeDtypeStruct((N_NODES, D), jnp.float32),
            jax.ShapeDtypeStruct((1, N_NODES), jnp.float32),
            jax.ShapeDtypeStruct((1, N_NODES), jnp.float32),
            jax.ShapeDtypeStruct((1, NZ), jnp.float32),
        ],
    )(x, w1, b1r, wg2, bgr, degp)


def _tc_b_body(hraw_ref, aggp_ref, wg_ref, bg_ref, h_ref, a_ref, b_ref):
    h = EPS * hraw_ref[...] + aggp_ref[0] + aggp_ref[1]
    h_ref[...] = h
    g2 = lax.dot_general(wg_ref[...], h, (((1,), (1,)), ((), ())),
                         preferred_element_type=jnp.float32)
    a_ref[...] = g2[0:1, :] + bg_ref[...]
    b_ref[...] = g2[1:2, :]


def _tc_b(hraw, aggp, wg2, bgr):
    return pl.pallas_call(
        _tc_b_body,
        grid=(N_NODES // _BN,),
        in_specs=[
            pl.BlockSpec((_BN, D), lambda i: (i, 0)),
            pl.BlockSpec((2, _BN, D), lambda i: (0, i, 0)),
            pl.BlockSpec((2, D), lambda i: (0, 0)),
            pl.BlockSpec((1, 1), lambda i: (0, 0)),
        ],
        out_specs=[
            pl.BlockSpec((_BN, D), lambda i: (i, 0)),
            pl.BlockSpec((1, _BN), lambda i: (0, i)),
            pl.BlockSpec((1, _BN), lambda i: (0, i)),
        ],
        out_shape=[
            jax.ShapeDtypeStruct((N_NODES, D), jnp.float32),
            jax.ShapeDtypeStruct((1, N_NODES), jnp.float32),
            jax.ShapeDtypeStruct((1, N_NODES), jnp.float32),
        ],
    )(hraw, aggp, wg2, bgr)


def _tc_c_body(hraw_ref, aggp_ref, w2_ref, b2_ref, out_ref):
    h = EPS * hraw_ref[...] + aggp_ref[0] + aggp_ref[1]
    logits = lax.dot_general(h, w2_ref[...], (((1,), (1,)), ((), ())),
                             preferred_element_type=jnp.float32)
    logits = logits + b2_ref[...]
    m = jnp.max(logits, axis=1, keepdims=True)
    z = logits - m
    lse = jnp.log(jnp.sum(jnp.exp(z), axis=1, keepdims=True))
    out_ref[...] = z - lse


def _tc_c(hraw, aggp, w2, b2r):
    ncls = w2.shape[0]
    return pl.pallas_call(
        _tc_c_body,
        grid=(N_NODES // _BN,),
        in_specs=[
            pl.BlockSpec((_BN, D), lambda i: (i, 0)),
            pl.BlockSpec((2, _BN, D), lambda i: (0, i, 0)),
            pl.BlockSpec((ncls, D), lambda i: (0, 0)),
            pl.BlockSpec((1, ncls), lambda i: (0, 0)),
        ],
        out_specs=pl.BlockSpec((_BN, ncls), lambda i: (i, 0)),
        out_shape=jax.ShapeDtypeStruct((N_NODES, ncls), jnp.float32),
    )(hraw, aggp, w2, b2r)


def kernel(x, edge_index, W1, b1, Wg0, bg0, Wg1, bg1, W2, b2):
    row = edge_index[0]
    col = edge_index[1]
    pad = EPT_PAD - EPT
    rowp = jnp.pad(row.reshape(NW, EPT), ((0, 0), (0, pad)),
                   constant_values=N_NODES).reshape(NW, NCHUNK, CH)
    colp = jnp.pad(col.reshape(NW, EPT), ((0, 0), (0, pad)),
                   constant_values=N_NODES).reshape(NW, NCHUNK, CH)
    z640 = jnp.zeros((640,), jnp.float32)
    zrows = jnp.zeros((626, D), jnp.float32)

    degp = _sc_deg(rowp, z640)
    h, ar, br, ndr = _tc_a(x, W1, b1.reshape(1, D), Wg0.reshape(2, D),
                           bg0.reshape(1, 1), degp)

    ndp = ndr[0, :NP]
    hp = jnp.pad(h, ((0, NP - N_NODES), (0, 0)))
    ap = jnp.pad(ar[0], (0, NP - N_NODES))
    bp = jnp.pad(br[0], (0, NP - N_NODES))
    agg1 = _sc_layer(hp, ap, bp, ndp, rowp, colp, zrows)

    h1, ar1, br1 = _tc_b(h, agg1, Wg1.reshape(2, D), bg1.reshape(1, 1))
    h1p = jnp.pad(h1, ((0, NP - N_NODES), (0, 0)))
    a1p = jnp.pad(ar1[0], (0, NP - N_NODES))
    b1p = jnp.pad(br1[0], (0, NP - N_NODES))
    agg2 = _sc_layer(h1p, a1p, b1p, ndp, rowp, colp, zrows)

    return _tc_c(h, agg2, W2, b2.reshape(1, 40))


# speedup vs baseline: 9.3513x; 9.3513x over previous
import functools

import jax
import jax.numpy as jnp
from jax import lax
from jax.experimental import pallas as pl
from jax.experimental.pallas import tpu as pltpu
from jax.experimental.pallas import tpu_sc as plsc

N_NODES = 10000
N_EDGES = 320000
D = 128
EPS = 0.3

NC = 2
NS = 16
NW = NC * NS
CH = 128
EPT = N_EDGES // NW
NCHUNK = (EPT + CH - 1) // CH
EPT_PAD = NCHUNK * CH
NP = N_NODES + 16
NZ = 10240

_mesh = plsc.VectorSubcoreMesh(core_axis_name="c", subcore_axis_name="s")


@functools.partial(
    pl.kernel,
    out_type=[jax.ShapeDtypeStruct((NZ,), jnp.float32),
              jax.ShapeDtypeStruct((NZ,), jnp.float32)],
    mesh=_mesh,
    compiler_params=pltpu.CompilerParams(needs_layout_passes=False),
    scratch_types=dict(
        deg=pltpu.VMEM_SHARED((NZ,), jnp.float32),
        rowb=pltpu.VMEM((1, CH), jnp.int32),
        vbuf=pltpu.VMEM((CH,), jnp.float32),
    ),
)
def _sc_deg(row_hbm, z640_hbm, out0_hbm, out1_hbm, deg, rowb, vbuf):
    c = lax.axis_index("c")
    s = lax.axis_index("s")
    w = c * NS + s
    pltpu.sync_copy(z640_hbm, deg.at[pl.ds(s * 640, 640)])

    def fill(k, _):
        vbuf[pl.ds(k * 16, 16)] = jnp.full((16,), 1.0, jnp.float32)
        return 0

    lax.fori_loop(0, CH // 16, fill, 0)
    plsc.subcore_barrier()

    def chunk(j, _):
        pltpu.sync_copy(row_hbm.at[w, j], rowb)
        pltpu.sync_copy(vbuf, deg.at[rowb.at[0]], add=True)
        return 0

    lax.fori_loop(0, NCHUNK, chunk, 0)
    plsc.subcore_barrier()

    @pl.when(c == 0)
    def _():
        pltpu.sync_copy(deg.at[pl.ds(s * 640, 640)], out0_hbm.at[pl.ds(s * 640, 640)])

    @pl.when(c == 1)
    def _():
        pltpu.sync_copy(deg.at[pl.ds(s * 640, 640)], out1_hbm.at[pl.ds(s * 640, 640)])


@functools.partial(
    pl.kernel,
    out_type=[jax.ShapeDtypeStruct((NS, 640, D), jnp.float32),
              jax.ShapeDtypeStruct((NS, 640, D), jnp.float32)],
    mesh=_mesh,
    compiler_params=pltpu.CompilerParams(needs_layout_passes=False),
    scratch_types=dict(
        agg=pltpu.VMEM_SHARED((NZ, D), jnp.float32),
        rowb=pltpu.VMEM((1, CH), jnp.int32),
        colb=pltpu.VMEM((1, CH), jnp.int32),
        av=pltpu.VMEM((NP,), jnp.float32),
        bv=pltpu.VMEM((NP,), jnp.float32),
        cbuf=pltpu.VMEM((CH,), jnp.float32),
        rows=pltpu.VMEM((CH, D), jnp.float32),
        sem=pltpu.SemaphoreType.DMA,
    ),
)
def _sc_layer(hh_hbm, a_hbm, b_hbm, row_hbm, col_hbm, zrows_hbm,
              out0_hbm, out1_hbm, agg, rowb, colb, av, bv, cbuf, rows, sem):
    c = lax.axis_index("c")
    s = lax.axis_index("s")
    w = c * NS + s
    pltpu.sync_copy(a_hbm, av)
    pltpu.sync_copy(b_hbm, bv)
    pltpu.sync_copy(zrows_hbm, agg.at[pl.ds(s * 640, 640)])
    plsc.subcore_barrier()

    def chunk(j, _):
        pltpu.sync_copy(row_hbm.at[w, j], rowb)
        pltpu.sync_copy(col_hbm.at[w, j], colb)

        def sub(k, _):
            ir = rowb[0, pl.ds(k * 16, 16)]
            ic = colb[0, pl.ds(k * 16, 16)]
            ar = plsc.load_gather(av, [ir])
            bc = plsc.load_gather(bv, [ic])
            t = ar + bc
            e = jnp.exp(-2.0 * jnp.abs(t))
            th = (1.0 - e) / (1.0 + e)
            cbuf[pl.ds(k * 16, 16)] = jnp.where(t < 0.0, -th, th)
            return 0

        lax.fori_loop(0, CH // 16, sub, 0)
        pltpu.async_copy(hh_hbm.at[rowb.at[0]], rows, sem).wait()

        def scale(g, _):
            cv = cbuf[pl.ds(g * 16, 16)]
            for l in range(16):
                e = g * 16 + l
                ce = cv[l]
                for q in range(D // 16):
                    rows[e, pl.ds(q * 16, 16)] = rows[e, pl.ds(q * 16, 16)] * ce
            return 0

        lax.fori_loop(0, CH // 16, scale, 0)
        pltpu.sync_copy(rows, agg.at[colb.at[0]], add=True)
        return 0

    lax.fori_loop(0, NCHUNK, chunk, 0)
    plsc.subcore_barrier()

    @pl.when(c == 0)
    def _():
        pltpu.sync_copy(agg.at[pl.ds(s * 640, 640)], out0_hbm.at[s])

    @pl.when(c == 1)
    def _():
        pltpu.sync_copy(agg.at[pl.ds(s * 640, 640)], out1_hbm.at[s])


def _tc_a_body(x_ref, w1_ref, b1_ref, wg_ref, bg_ref, d0_ref, d1_ref,
               h_ref, hh_ref, a_ref, b_ref, nd_ref):
    nd = lax.rsqrt(jnp.maximum(d0_ref[...] + d1_ref[...], 1.0))
    nd_ref[...] = nd
    h = lax.dot_general(x_ref[...], w1_ref[...], (((1,), (1,)), ((), ())),
                        preferred_element_type=jnp.float32)
    h = jnp.maximum(h + b1_ref[...], 0.0)
    h_ref[...] = h
    hh_ref[...] = h * nd[:N_NODES, :]
    g2 = lax.dot_general(wg_ref[...], h, (((1,), (1,)), ((), ())),
                         preferred_element_type=jnp.float32)
    a_ref[...] = g2[0:1, :] + bg_ref[...]
    b_ref[...] = g2[1:2, :]


def _tc_a(x, w1, b1r, wg2, bgr, d0, d1):
    return pl.pallas_call(
        _tc_a_body,
        out_shape=[
            jax.ShapeDtypeStruct((N_NODES, D), jnp.float32),
            jax.ShapeDtypeStruct((N_NODES, D), jnp.float32),
            jax.ShapeDtypeStruct((1, N_NODES), jnp.float32),
            jax.ShapeDtypeStruct((1, N_NODES), jnp.float32),
            jax.ShapeDtypeStruct((NZ, 1), jnp.float32),
        ],
    )(x, w1, b1r, wg2, bgr, d0, d1)


def _tc_b_body(hraw_ref, agg0_ref, agg1_ref, nd_ref, wg_ref, bg_ref,
               h_ref, hh_ref, a_ref, b_ref):
    nd = nd_ref[:N_NODES, :]
    h = EPS * hraw_ref[...] + nd * (agg0_ref[:N_NODES, :]
                                    + agg1_ref[:N_NODES, :])
    h_ref[...] = h
    hh_ref[...] = h * nd
    g2 = lax.dot_general(wg_ref[...], h, (((1,), (1,)), ((), ())),
                         preferred_element_type=jnp.float32)
    a_ref[...] = g2[0:1, :] + bg_ref[...]
    b_ref[...] = g2[1:2, :]


def _tc_b(hraw, agg0, agg1, ndc, wg2, bgr):
    return pl.pallas_call(
        _tc_b_body,
        out_shape=[
            jax.ShapeDtypeStruct((N_NODES, D), jnp.float32),
            jax.ShapeDtypeStruct((N_NODES, D), jnp.float32),
            jax.ShapeDtypeStruct((1, N_NODES), jnp.float32),
            jax.ShapeDtypeStruct((1, N_NODES), jnp.float32),
        ],
    )(hraw, agg0, agg1, ndc, wg2, bgr)


def _tc_c_body(hraw_ref, agg0_ref, agg1_ref, nd_ref, w2_ref, b2_ref, out_ref):
    nd = nd_ref[:N_NODES, :]
    h = EPS * hraw_ref[...] + nd * (agg0_ref[:N_NODES, :]
                                    + agg1_ref[:N_NODES, :])
    logits = lax.dot_general(h, w2_ref[...], (((1,), (1,)), ((), ())),
                             preferred_element_type=jnp.float32)
    logits = logits + b2_ref[...]
    m = jnp.max(logits, axis=1, keepdims=True)
    z = logits - m
    lse = jnp.log(jnp.sum(jnp.exp(z), axis=1, keepdims=True))
    out_ref[...] = z - lse


def _tc_c(hraw, agg0, agg1, ndc, w2, b2r):
    ncls = w2.shape[0]
    return pl.pallas_call(
        _tc_c_body,
        out_shape=jax.ShapeDtypeStruct((N_NODES, ncls), jnp.float32),
    )(hraw, agg0, agg1, ndc, w2, b2r)


def kernel(x, edge_index, W1, b1, Wg0, bg0, Wg1, bg1, W2, b2):
    row = edge_index[0]
    col = edge_index[1]
    pad = EPT_PAD - EPT
    rowp = jnp.pad(row.reshape(NW, EPT), ((0, 0), (0, pad)),
                   constant_values=N_NODES).reshape(NW, NCHUNK, 1, CH)
    colp = jnp.pad(col.reshape(NW, EPT), ((0, 0), (0, pad)),
                   constant_values=N_NODES).reshape(NW, NCHUNK, 1, CH)
    z640 = jnp.zeros((640,), jnp.float32)
    zrows = jnp.zeros((640, D), jnp.float32)

    deg0, deg1 = _sc_deg(rowp, z640)
    h, hh, ar, br, ndc = _tc_a(x, W1, b1.reshape(1, D), Wg0.reshape(2, D),
                               bg0.reshape(1, 1), deg0.reshape(NZ, 1),
                               deg1.reshape(NZ, 1))

    hhp = jnp.pad(hh, ((0, NP - N_NODES), (0, 0)))
    ap = jnp.pad(ar[0], (0, NP - N_NODES))
    bp = jnp.pad(br[0], (0, NP - N_NODES))
    a0, a1 = _sc_layer(hhp, ap, bp, rowp, colp, zrows)

    h1, hh1, ar1, br1 = _tc_b(h, a0.reshape(NZ, D), a1.reshape(NZ, D), ndc,
                              Wg1.reshape(2, D), bg1.reshape(1, 1))
    hh1p = jnp.pad(hh1, ((0, NP - N_NODES), (0, 0)))
    a1p = jnp.pad(ar1[0], (0, NP - N_NODES))
    b1p = jnp.pad(br1[0], (0, NP - N_NODES))
    c0, c1 = _sc_layer(hh1p, a1p, b1p, rowp, colp, zrows)

    return _tc_c(h, c0.reshape(NZ, D), c1.reshape(NZ, D), ndc, W2,
                 b2.reshape(1, 40))


# speedup vs baseline: 9.8083x; 1.0489x over previous
import functools

import jax
import jax.numpy as jnp
from jax import lax
from jax.experimental import pallas as pl
from jax.experimental.pallas import tpu as pltpu
from jax.experimental.pallas import tpu_sc as plsc

N_NODES = 10000
N_EDGES = 320000
D = 128
EPS = 0.3

NC = 2
NS = 16
NW = NC * NS
CH = 128
EPT = N_EDGES // NW
NCHUNK = 80
EPT_PAD = NCHUNK * CH
NP = N_NODES + 16
NZ = 10240

_mesh = plsc.VectorSubcoreMesh(core_axis_name="c", subcore_axis_name="s")


@functools.partial(
    pl.kernel,
    out_type=[jax.ShapeDtypeStruct((NZ,), jnp.float32),
              jax.ShapeDtypeStruct((NZ,), jnp.float32)],
    mesh=_mesh,
    compiler_params=pltpu.CompilerParams(needs_layout_passes=False),
    scratch_types=dict(
        deg=pltpu.VMEM_SHARED((NZ,), jnp.float32),
        rowi=pltpu.VMEM((NCHUNK // 2, 1, CH), jnp.int32),
        rowu=pltpu.VMEM((NCHUNK, 1, CH), jnp.int32),
        vbuf=pltpu.VMEM((CH,), jnp.float32),
        sem=pltpu.SemaphoreType.DMA,
    ),
)
def _sc_deg(row_hbm, z640_hbm, out0_hbm, out1_hbm, deg, rowi, rowu, vbuf,
            sem):
    c = lax.axis_index("c")
    s = lax.axis_index("s")
    w = c * NS + s
    pltpu.sync_copy(row_hbm.at[w], rowi)
    pltpu.sync_copy(z640_hbm, deg.at[pl.ds(s * 640, 640)])

    def fill(k, _):
        vbuf[pl.ds(k * 16, 16)] = jnp.full((16,), 1.0, jnp.float32)
        return 0

    lax.fori_loop(0, CH // 16, fill, 0)

    def unpack(j, _):
        jw = j // 2
        off = (j % 2) * (CH // 2)

        def up(g, _):
            vr = rowi[jw, 0, pl.ds(off + g * 16, 16)]
            rowu[j, 0, pl.ds(g * 32, 16)] = vr & 0xFFFF
            rowu[j, 0, pl.ds(g * 32 + 16, 16)] = lax.shift_right_logical(vr, 16)
            return 0

        lax.fori_loop(0, CH // 32, up, 0)
        return 0

    lax.fori_loop(0, NCHUNK, unpack, 0)
    plsc.subcore_barrier()

    DEPTH = 8
    for j in range(DEPTH):
        pltpu.async_copy(vbuf, deg.at[rowu.at[j, 0]], sem, add=True)

    def chunk(j, _):
        pltpu.make_async_copy(vbuf, deg.at[rowu.at[0, 0]], sem).wait()
        pltpu.async_copy(vbuf, deg.at[rowu.at[j, 0]], sem, add=True)
        return 0

    lax.fori_loop(DEPTH, NCHUNK, chunk, 0)
    for j in range(DEPTH):
        pltpu.make_async_copy(vbuf, deg.at[rowu.at[0, 0]], sem).wait()
    plsc.subcore_barrier()

    @pl.when(c == 0)
    def _():
        pltpu.sync_copy(deg.at[pl.ds(s * 640, 640)], out0_hbm.at[pl.ds(s * 640, 640)])

    @pl.when(c == 1)
    def _():
        pltpu.sync_copy(deg.at[pl.ds(s * 640, 640)], out1_hbm.at[pl.ds(s * 640, 640)])


@functools.partial(
    pl.kernel,
    out_type=[jax.ShapeDtypeStruct((NS, 640, D), jnp.float32),
              jax.ShapeDtypeStruct((NS, 640, D), jnp.float32)],
    mesh=_mesh,
    compiler_params=pltpu.CompilerParams(needs_layout_passes=False),
    scratch_types=dict(
        agg=pltpu.VMEM_SHARED((NZ, D), jnp.float32),
        rowi=pltpu.VMEM((NCHUNK // 2, 1, CH), jnp.int32),
        coli=pltpu.VMEM((NCHUNK // 2, 1, CH), jnp.int32),
        rowb=pltpu.VMEM((2, 1, CH), jnp.int32),
        colb=pltpu.VMEM((2, 1, CH), jnp.int32),
        ab=pltpu.VMEM((2, 2, CH), jnp.float32),
        rows0=pltpu.VMEM((CH, D), jnp.float32),
        rows1=pltpu.VMEM((CH, D), jnp.float32),
        sab0=pltpu.SemaphoreType.DMA,
        sab1=pltpu.SemaphoreType.DMA,
        sr0=pltpu.SemaphoreType.DMA,
        sr1=pltpu.SemaphoreType.DMA,
        ss0=pltpu.SemaphoreType.DMA,
        ss1=pltpu.SemaphoreType.DMA,
    ),
)
def _sc_layer(hh_hbm, a_hbm, b_hbm, row_hbm, col_hbm, zrows_hbm,
              out0_hbm, out1_hbm, agg, rowi, coli, rowb, colb, ab,
              rows0, rows1, sab0, sab1, sr0, sr1, ss0, ss1):
    c = lax.axis_index("c")
    s = lax.axis_index("s")
    w = c * NS + s
    pltpu.sync_copy(row_hbm.at[w], rowi)
    pltpu.sync_copy(col_hbm.at[w], coli)
    pltpu.sync_copy(zrows_hbm, agg.at[pl.ds(s * 640, 640)])
    plsc.subcore_barrier()

    rows_ = (rows0, rows1)
    sab_ = (sab0, sab1)
    sr_ = (sr0, sr1)
    ss_ = (ss0, ss1)

    def unpack_issue_gathers(j, p):
        jw = j // 2
        off = (j % 2) * (CH // 2)

        def up(g, _):
            vr = rowi[jw, 0, pl.ds(off + g * 16, 16)]
            vc = coli[jw, 0, pl.ds(off + g * 16, 16)]
            rowb[p, 0, pl.ds(g * 32, 16)] = vr & 0xFFFF
            rowb[p, 0, pl.ds(g * 32 + 16, 16)] = lax.shift_right_logical(vr, 16)
            colb[p, 0, pl.ds(g * 32, 16)] = vc & 0xFFFF
            colb[p, 0, pl.ds(g * 32 + 16, 16)] = lax.shift_right_logical(vc, 16)
            return 0

        lax.fori_loop(0, CH // 32, up, 0)
        pltpu.async_copy(a_hbm.at[rowb.at[p, 0]], ab.at[p, 0], sab_[p])
        pltpu.async_copy(b_hbm.at[colb.at[p, 0]], ab.at[p, 1], sab_[p])
        pltpu.async_copy(hh_hbm.at[rowb.at[p, 0]], rows_[p], sr_[p])

    def wait_gathers(p):
        pltpu.make_async_copy(a_hbm.at[rowb.at[p, 0]], ab.at[p, 0], sab_[p]).wait()
        pltpu.make_async_copy(b_hbm.at[colb.at[p, 0]], ab.at[p, 1], sab_[p]).wait()
        pltpu.make_async_copy(hh_hbm.at[rowb.at[p, 0]], rows_[p], sr_[p]).wait()

    def wait_scatter(p):
        pltpu.make_async_copy(rows_[p], agg.at[colb.at[p, 0]], ss_[p]).wait()

    def compute(j, p):
        def group(g, _):
            t = ab[p, 0, pl.ds(g * 16, 16)] + ab[p, 1, pl.ds(g * 16, 16)]
            e = jnp.exp(-2.0 * jnp.abs(t))
            th = (1.0 - e) / (1.0 + e)
            th = jnp.where(t < 0.0, -th, th)
            r = rows_[p]
            for l in range(16):
                ce = th[l]
                ei = g * 16 + l
                for q in range(D // 16):
                    r[ei, pl.ds(q * 16, 16)] = r[ei, pl.ds(q * 16, 16)] * ce
            return 0

        lax.fori_loop(0, CH // 16, group, 0)
        pltpu.async_copy(rows_[p], agg.at[colb.at[p, 0]], ss_[p], add=True)

    unpack_issue_gathers(0, 0)
    unpack_issue_gathers(1, 1)
    wait_gathers(0)
    compute(0, 0)

    def body(i, _):
        j = 2 * i + 1
        wait_scatter(0)
        unpack_issue_gathers(j + 1, 0)
        wait_gathers(1)
        compute(j, 1)
        wait_scatter(1)
        unpack_issue_gathers(j + 2, 1)
        wait_gathers(0)
        compute(j + 1, 0)
        return 0

    lax.fori_loop(0, NCHUNK // 2 - 2, body, 0)
    j = NCHUNK - 3
    wait_scatter(0)
    unpack_issue_gathers(j + 1, 0)
    wait_gathers(1)
    compute(j, 1)
    wait_scatter(1)
    unpack_issue_gathers(j + 2, 1)
    wait_gathers(0)
    compute(j + 1, 0)
    wait_gathers(1)
    compute(j + 2, 1)
    wait_scatter(0)
    wait_scatter(1)

    plsc.subcore_barrier()

    @pl.when(c == 0)
    def _():
        pltpu.sync_copy(agg.at[pl.ds(s * 640, 640)], out0_hbm.at[s])

    @pl.when(c == 1)
    def _():
        pltpu.sync_copy(agg.at[pl.ds(s * 640, 640)], out1_hbm.at[s])


def _tc_a_body(x_ref, w1_ref, b1_ref, wg_ref, bg_ref, d0_ref, d1_ref,
               h_ref, hh_ref, a_ref, b_ref, nd_ref):
    nd = lax.rsqrt(jnp.maximum(d0_ref[...] + d1_ref[...], 1.0))
    nd_ref[...] = nd
    h = lax.dot_general(x_ref[...], w1_ref[...], (((1,), (1,)), ((), ())),
                        preferred_element_type=jnp.float32)
    h = jnp.maximum(h + b1_ref[...], 0.0)
    h_ref[...] = h
    hh_ref[:N_NODES, :] = h * nd[:N_NODES, :]
    g2 = lax.dot_general(wg_ref[...], h, (((1,), (1,)), ((), ())),
                         preferred_element_type=jnp.float32)
    a_ref[...] = g2[0:1, :] + bg_ref[...]
    b_ref[...] = g2[1:2, :]


def _tc_a(x, w1, b1r, wg2, bgr, d0, d1):
    return pl.pallas_call(
        _tc_a_body,
        out_shape=[
            jax.ShapeDtypeStruct((N_NODES, D), jnp.float32),
            jax.ShapeDtypeStruct((NP, D), jnp.float32),
            jax.ShapeDtypeStruct((1, N_NODES), jnp.float32),
            jax.ShapeDtypeStruct((1, N_NODES), jnp.float32),
            jax.ShapeDtypeStruct((NZ, 1), jnp.float32),
        ],
    )(x, w1, b1r, wg2, bgr, d0, d1)


def _tc_b_body(hraw_ref, agg0_ref, agg1_ref, nd_ref, wg_ref, bg_ref,
               h_ref, hh_ref, a_ref, b_ref):
    nd = nd_ref[:N_NODES, :]
    h = EPS * hraw_ref[...] + nd * (agg0_ref[:N_NODES, :]
                                    + agg1_ref[:N_NODES, :])
    h_ref[...] = h
    hh_ref[:N_NODES, :] = h * nd
    g2 = lax.dot_general(wg_ref[...], h, (((1,), (1,)), ((), ())),
                         preferred_element_type=jnp.float32)
    a_ref[...] = g2[0:1, :] + bg_ref[...]
    b_ref[...] = g2[1:2, :]


def _tc_b(hraw, agg0, agg1, ndc, wg2, bgr):
    return pl.pallas_call(
        _tc_b_body,
        out_shape=[
            jax.ShapeDtypeStruct((N_NODES, D), jnp.float32),
            jax.ShapeDtypeStruct((NP, D), jnp.float32),
            jax.ShapeDtypeStruct((1, N_NODES), jnp.float32),
            jax.ShapeDtypeStruct((1, N_NODES), jnp.float32),
        ],
    )(hraw, agg0, agg1, ndc, wg2, bgr)


def _tc_c_body(hraw_ref, agg0_ref, agg1_ref, nd_ref, w2_ref, b2_ref, out_ref):
    nd = nd_ref[:N_NODES, :]
    h = EPS * hraw_ref[...] + nd * (agg0_ref[:N_NODES, :]
                                    + agg1_ref[:N_NODES, :])
    logits = lax.dot_general(h, w2_ref[...], (((1,), (1,)), ((), ())),
                             preferred_element_type=jnp.float32)
    logits = logits + b2_ref[...]
    m = jnp.max(logits, axis=1, keepdims=True)
    z = logits - m
    lse = jnp.log(jnp.sum(jnp.exp(z), axis=1, keepdims=True))
    out_ref[...] = z - lse


def _tc_c(hraw, agg0, agg1, ndc, w2, b2r):
    ncls = w2.shape[0]
    return pl.pallas_call(
        _tc_c_body,
        out_shape=jax.ShapeDtypeStruct((N_NODES, ncls), jnp.float32),
    )(hraw, agg0, agg1, ndc, w2, b2r)


def kernel(x, edge_index, W1, b1, Wg0, bg0, Wg1, bg1, W2, b2):
    row = edge_index[0]
    col = edge_index[1]
    pad = EPT_PAD - EPT
    def pack_u16(v):
        v = jnp.pad(v.reshape(NW, EPT), ((0, 0), (0, pad)),
                    constant_values=N_NODES)
        v = v.reshape(NW, NCHUNK, CH // 32, 2, 16)
        packed = v[:, :, :, 0, :] | (v[:, :, :, 1, :] << 16)
        return packed.reshape(NW, NCHUNK // 2, 1, CH)

    rowp = pack_u16(row)
    colp = pack_u16(col)
    z640 = jnp.zeros((640,), jnp.float32)
    zrows = jnp.zeros((640, D), jnp.float32)

    deg0, deg1 = _sc_deg(rowp, z640)
    h, hh, ar, br, ndc = _tc_a(x, W1, b1.reshape(1, D), Wg0.reshape(2, D),
                               bg0.reshape(1, 1), deg0.reshape(NZ, 1),
                               deg1.reshape(NZ, 1))

    ap = jnp.pad(ar.reshape(N_NODES), (0, NP - N_NODES))
    bp = jnp.pad(br.reshape(N_NODES), (0, NP - N_NODES))
    a0, a1 = _sc_layer(hh, ap, bp, rowp, colp, zrows)

    h1, hh1, ar1, br1 = _tc_b(h, a0.reshape(NZ, D), a1.reshape(NZ, D), ndc,
                              Wg1.reshape(2, D), bg1.reshape(1, 1))
    a1p = jnp.pad(ar1.reshape(N_NODES), (0, NP - N_NODES))
    b1p = jnp.pad(br1.reshape(N_NODES), (0, NP - N_NODES))
    c0, c1 = _sc_layer(hh1, a1p, b1p, rowp, colp, zrows)

    return _tc_c(h, c0.reshape(NZ, D), c1.reshape(NZ, D), ndc, W2,
                 b2.reshape(1, 40))


# speedup vs baseline: 11.7110x; 1.1940x over previous
import functools

import jax
import jax.numpy as jnp
from jax import lax
from jax.experimental import pallas as pl
from jax.experimental.pallas import tpu as pltpu
from jax.experimental.pallas import tpu_sc as plsc

N_NODES = 10000
N_EDGES = 320000
D = 128
EPS = 0.3

NC = 2
NS = 16
NW = NC * NS
CH = 64
EPT = N_EDGES // NW
NCHUNK = 160
NPROC = 159
EPT_PAD = NCHUNK * CH
NP = N_NODES + 16
NZ = 10240

_mesh = plsc.VectorSubcoreMesh(core_axis_name="c", subcore_axis_name="s")


@functools.partial(
    pl.kernel,
    out_type=[jax.ShapeDtypeStruct((NZ,), jnp.float32),
              jax.ShapeDtypeStruct((NZ,), jnp.float32)],
    mesh=_mesh,
    compiler_params=pltpu.CompilerParams(needs_layout_passes=False),
    scratch_types=dict(
        deg=pltpu.VMEM_SHARED((NZ,), jnp.float32),
        rowi=pltpu.VMEM((NCHUNK // 4, 1, 128), jnp.int32),
        rowu=pltpu.VMEM((NCHUNK, 1, CH), jnp.int32),
        vbuf=pltpu.VMEM((CH,), jnp.float32),
        sem=pltpu.SemaphoreType.DMA,
    ),
)
def _sc_deg(row_hbm, z640_hbm, out0_hbm, out1_hbm, deg, rowi, rowu, vbuf,
            sem):
    c = lax.axis_index("c")
    s = lax.axis_index("s")
    w = c * NS + s
    pltpu.sync_copy(row_hbm.at[w], rowi)
    pltpu.sync_copy(z640_hbm, deg.at[pl.ds(s * 640, 640)])

    def fill(k, _):
        vbuf[pl.ds(k * 16, 16)] = jnp.full((16,), 1.0, jnp.float32)
        return 0

    lax.fori_loop(0, CH // 16, fill, 0)

    def unpack(j, _):
        jw = j // 4
        off = (j % 4) * (CH // 2)

        def up(g, _):
            vr = rowi[jw, 0, pl.ds(off + g * 16, 16)]
            rowu[j, 0, pl.ds(g * 32, 16)] = vr & 0xFFFF
            rowu[j, 0, pl.ds(g * 32 + 16, 16)] = lax.shift_right_logical(vr, 16)
            return 0

        lax.fori_loop(0, CH // 32, up, 0)
        return 0

    lax.fori_loop(0, NCHUNK, unpack, 0)
    plsc.subcore_barrier()

    DEPTH = 8
    for j in range(DEPTH):
        pltpu.async_copy(vbuf, deg.at[rowu.at[j, 0]], sem, add=True)

    def chunk(j, _):
        pltpu.make_async_copy(vbuf, deg.at[rowu.at[0, 0]], sem).wait()
        pltpu.async_copy(vbuf, deg.at[rowu.at[j, 0]], sem, add=True)
        return 0

    lax.fori_loop(DEPTH, NCHUNK, chunk, 0)
    for j in range(DEPTH):
        pltpu.make_async_copy(vbuf, deg.at[rowu.at[0, 0]], sem).wait()
    plsc.subcore_barrier()

    @pl.when(c == 0)
    def _():
        pltpu.sync_copy(deg.at[pl.ds(s * 640, 640)], out0_hbm.at[pl.ds(s * 640, 640)])

    @pl.when(c == 1)
    def _():
        pltpu.sync_copy(deg.at[pl.ds(s * 640, 640)], out1_hbm.at[pl.ds(s * 640, 640)])


@functools.partial(
    pl.kernel,
    out_type=[jax.ShapeDtypeStruct((NS, 640, D), jnp.float32),
              jax.ShapeDtypeStruct((NS, 640, D), jnp.float32)],
    mesh=_mesh,
    compiler_params=pltpu.CompilerParams(needs_layout_passes=False),
    scratch_types=dict(
        agg=pltpu.VMEM_SHARED((NZ, D), jnp.float32),
        rowi=pltpu.VMEM((NCHUNK // 4, 1, 128), jnp.int32),
        coli=pltpu.VMEM((NCHUNK // 4, 1, 128), jnp.int32),
        rowb=pltpu.VMEM((3, 1, CH), jnp.int32),
        colb=pltpu.VMEM((3, 1, CH), jnp.int32),
        ab=pltpu.VMEM((3, 2, CH), jnp.float32),
        rows0=pltpu.VMEM((CH, D), jnp.float32),
        rows1=pltpu.VMEM((CH, D), jnp.float32),
        rows2=pltpu.VMEM((CH, D), jnp.float32),
        sab0=pltpu.SemaphoreType.DMA,
        sab1=pltpu.SemaphoreType.DMA,
        sab2=pltpu.SemaphoreType.DMA,
        sr0=pltpu.SemaphoreType.DMA,
        sr1=pltpu.SemaphoreType.DMA,
        sr2=pltpu.SemaphoreType.DMA,
        ss0=pltpu.SemaphoreType.DMA,
        ss1=pltpu.SemaphoreType.DMA,
        ss2=pltpu.SemaphoreType.DMA,
    ),
)
def _sc_layer(hh_hbm, a_hbm, b_hbm, row_hbm, col_hbm, zrows_hbm,
              out0_hbm, out1_hbm, agg, rowi, coli, rowb, colb, ab,
              rows0, rows1, rows2,
              sab0, sab1, sab2, sr0, sr1, sr2, ss0, ss1, ss2):
    c = lax.axis_index("c")
    s = lax.axis_index("s")
    w = c * NS + s
    pltpu.sync_copy(row_hbm.at[w], rowi)
    pltpu.sync_copy(col_hbm.at[w], coli)
    pltpu.sync_copy(zrows_hbm, agg.at[pl.ds(s * 640, 640)])
    plsc.subcore_barrier()

    rows_ = (rows0, rows1, rows2)
    sab_ = (sab0, sab1, sab2)
    sr_ = (sr0, sr1, sr2)
    ss_ = (ss0, ss1, ss2)

    def unpack_issue_gathers(j, k):
        jw = j // 4
        off = (j % 4) * (CH // 2)

        def up(g, _):
            vr = rowi[jw, 0, pl.ds(off + g * 16, 16)]
            vc = coli[jw, 0, pl.ds(off + g * 16, 16)]
            rowb[k, 0, pl.ds(g * 32, 16)] = vr & 0xFFFF
            rowb[k, 0, pl.ds(g * 32 + 16, 16)] = lax.shift_right_logical(vr, 16)
            colb[k, 0, pl.ds(g * 32, 16)] = vc & 0xFFFF
            colb[k, 0, pl.ds(g * 32 + 16, 16)] = lax.shift_right_logical(vc, 16)
            return 0

        lax.fori_loop(0, CH // 32, up, 0)
        pltpu.async_copy(a_hbm.at[rowb.at[k, 0]], ab.at[k, 0], sab_[k])
        pltpu.async_copy(b_hbm.at[colb.at[k, 0]], ab.at[k, 1], sab_[k])
        pltpu.async_copy(hh_hbm.at[rowb.at[k, 0]], rows_[k], sr_[k])

    def wait_gathers(k):
        pltpu.make_async_copy(a_hbm.at[rowb.at[k, 0]], ab.at[k, 0], sab_[k]).wait()
        pltpu.make_async_copy(b_hbm.at[colb.at[k, 0]], ab.at[k, 1], sab_[k]).wait()
        pltpu.make_async_copy(hh_hbm.at[rowb.at[k, 0]], rows_[k], sr_[k]).wait()

    def wait_scatter(k):
        pltpu.make_async_copy(rows_[k], agg.at[colb.at[k, 0]], ss_[k]).wait()

    def compute(j, k):
        def group(g, _):
            t = ab[k, 0, pl.ds(g * 16, 16)] + ab[k, 1, pl.ds(g * 16, 16)]
            e = jnp.exp(-2.0 * jnp.abs(t))
            th = (1.0 - e) / (1.0 + e)
            th = jnp.where(t < 0.0, -th, th)
            r = rows_[k]
            for l in range(16):
                ce = th[l]
                ei = g * 16 + l
                for q in range(D // 16):
                    r[ei, pl.ds(q * 16, 16)] = r[ei, pl.ds(q * 16, 16)] * ce
            return 0

        lax.fori_loop(0, CH // 16, group, 0)
        pltpu.async_copy(rows_[k], agg.at[colb.at[k, 0]], ss_[k], add=True)

    def step(j, k, kn, wait_sc, issue):
        if wait_sc:
            wait_scatter(kn)
        if issue:
            unpack_issue_gathers(j + 1, kn)
        wait_gathers(k)
        compute(j, k)

    unpack_issue_gathers(0, 0)
    step(0, 0, 1, False, True)
    step(1, 1, 2, False, True)
    step(2, 2, 0, True, True)

    def body(i, _):
        j = 3 * i
        step(j, 0, 1, True, True)
        step(j + 1, 1, 2, True, True)
        step(j + 2, 2, 0, True, True)
        return 0

    lax.fori_loop(1, NPROC // 3 - 1, body, 0)
    j = NPROC - 3
    step(j, 0, 1, True, True)
    step(j + 1, 1, 2, True, True)
    step(j + 2, 2, 0, True, False)
    wait_scatter(1)
    wait_scatter(2)

    plsc.subcore_barrier()

    @pl.when(c == 0)
    def _():
        pltpu.sync_copy(agg.at[pl.ds(s * 640, 640)], out0_hbm.at[s])

    @pl.when(c == 1)
    def _():
        pltpu.sync_copy(agg.at[pl.ds(s * 640, 640)], out1_hbm.at[s])


def _tc_a_body(x_ref, w1_ref, b1_ref, wg_ref, bg_ref, d0_ref, d1_ref,
               h_ref, hh_ref, a_ref, b_ref, nd_ref):
    nd = lax.rsqrt(jnp.maximum(d0_ref[...] + d1_ref[...], 1.0))
    nd_ref[...] = nd
    h = lax.dot_general(x_ref[...], w1_ref[...], (((1,), (1,)), ((), ())),
                        preferred_element_type=jnp.float32)
    h = jnp.maximum(h + b1_ref[...], 0.0)
    h_ref[...] = h
    hh_ref[:N_NODES, :] = h * nd[:N_NODES, :]
    g2 = lax.dot_general(wg_ref[...], h, (((1,), (1,)), ((), ())),
                         preferred_element_type=jnp.float32)
    a_ref[...] = g2[0:1, :] + bg_ref[...]
    b_ref[...] = g2[1:2, :]


def _tc_a(x, w1, b1r, wg2, bgr, d0, d1):
    return pl.pallas_call(
        _tc_a_body,
        out_shape=[
            jax.ShapeDtypeStruct((N_NODES, D), jnp.float32),
            jax.ShapeDtypeStruct((NP, D), jnp.float32),
            jax.ShapeDtypeStruct((1, N_NODES), jnp.float32),
            jax.ShapeDtypeStruct((1, N_NODES), jnp.float32),
            jax.ShapeDtypeStruct((NZ, 1), jnp.float32),
        ],
    )(x, w1, b1r, wg2, bgr, d0, d1)


def _tc_b_body(hraw_ref, agg0_ref, agg1_ref, nd_ref, wg_ref, bg_ref,
               h_ref, hh_ref, a_ref, b_ref):
    nd = nd_ref[:N_NODES, :]
    h = EPS * hraw_ref[...] + nd * (agg0_ref[:N_NODES, :]
                                    + agg1_ref[:N_NODES, :])
    h_ref[...] = h
    hh_ref[:N_NODES, :] = h * nd
    g2 = lax.dot_general(wg_ref[...], h, (((1,), (1,)), ((), ())),
                         preferred_element_type=jnp.float32)
    a_ref[...] = g2[0:1, :] + bg_ref[...]
    b_ref[...] = g2[1:2, :]


def _tc_b(hraw, agg0, agg1, ndc, wg2, bgr):
    return pl.pallas_call(
        _tc_b_body,
        out_shape=[
            jax.ShapeDtypeStruct((N_NODES, D), jnp.float32),
            jax.ShapeDtypeStruct((NP, D), jnp.float32),
            jax.ShapeDtypeStruct((1, N_NODES), jnp.float32),
            jax.ShapeDtypeStruct((1, N_NODES), jnp.float32),
        ],
    )(hraw, agg0, agg1, ndc, wg2, bgr)


def _tc_c_body(hraw_ref, agg0_ref, agg1_ref, nd_ref, w2_ref, b2_ref, out_ref):
    nd = nd_ref[:N_NODES, :]
    h = EPS * hraw_ref[...] + nd * (agg0_ref[:N_NODES, :]
                                    + agg1_ref[:N_NODES, :])
    logits = lax.dot_general(h, w2_ref[...], (((1,), (1,)), ((), ())),
                             preferred_element_type=jnp.float32)
    logits = logits + b2_ref[...]
    m = jnp.max(logits, axis=1, keepdims=True)
    z = logits - m
    lse = jnp.log(jnp.sum(jnp.exp(z), axis=1, keepdims=True))
    out_ref[...] = z - lse


def _tc_c(hraw, agg0, agg1, ndc, w2, b2r):
    ncls = w2.shape[0]
    return pl.pallas_call(
        _tc_c_body,
        out_shape=jax.ShapeDtypeStruct((N_NODES, ncls), jnp.float32),
    )(hraw, agg0, agg1, ndc, w2, b2r)


def kernel(x, edge_index, W1, b1, Wg0, bg0, Wg1, bg1, W2, b2):
    row = edge_index[0]
    col = edge_index[1]
    pad = EPT_PAD - EPT
    def pack_u16(v):
        v = jnp.pad(v.reshape(NW, EPT), ((0, 0), (0, pad)),
                    constant_values=N_NODES)
        v = v.reshape(NW, NCHUNK, CH // 32, 2, 16)
        packed = v[:, :, :, 0, :] | (v[:, :, :, 1, :] << 16)
        return packed.reshape(NW, NCHUNK // 4, 1, 128)

    rowp = pack_u16(row)
    colp = pack_u16(col)
    z640 = jnp.zeros((640,), jnp.float32)
    zrows = jnp.zeros((640, D), jnp.float32)

    deg0, deg1 = _sc_deg(rowp, z640)
    h, hh, ar, br, ndc = _tc_a(x, W1, b1.reshape(1, D), Wg0.reshape(2, D),
                               bg0.reshape(1, 1), deg0.reshape(NZ, 1),
                               deg1.reshape(NZ, 1))

    ap = jnp.pad(ar.reshape(N_NODES), (0, NP - N_NODES))
    bp = jnp.pad(br.reshape(N_NODES), (0, NP - N_NODES))
    a0, a1 = _sc_layer(hh, ap, bp, rowp, colp, zrows)

    h1, hh1, ar1, br1 = _tc_b(h, a0.reshape(NZ, D), a1.reshape(NZ, D), ndc,
                              Wg1.reshape(2, D), bg1.reshape(1, 1))
    a1p = jnp.pad(ar1.reshape(N_NODES), (0, NP - N_NODES))
    b1p = jnp.pad(br1.reshape(N_NODES), (0, NP - N_NODES))
    c0, c1 = _sc_layer(hh1, a1p, b1p, rowp, colp, zrows)

    return _tc_c(h, c0.reshape(NZ, D), c1.reshape(NZ, D), ndc, W2,
                 b2.reshape(1, 40))


# speedup vs baseline: 11.7297x; 1.0016x over previous
import functools

import jax
import jax.numpy as jnp
from jax import lax
from jax.experimental import pallas as pl
from jax.experimental.pallas import tpu as pltpu
from jax.experimental.pallas import tpu_sc as plsc

N_NODES = 10000
N_EDGES = 320000
D = 128
EPS = 0.3

NC = 2
NS = 16
NW = NC * NS
CH = 64
EPT = N_EDGES // NW
NCHUNK = 160
NPROC = 159
EPT_PAD = NCHUNK * CH
NP = N_NODES + 16
NZ = 10240

_mesh = plsc.VectorSubcoreMesh(core_axis_name="c", subcore_axis_name="s")


@functools.partial(
    pl.kernel,
    out_type=[jax.ShapeDtypeStruct((NZ,), jnp.float32),
              jax.ShapeDtypeStruct((NZ,), jnp.float32)],
    mesh=_mesh,
    compiler_params=pltpu.CompilerParams(needs_layout_passes=False),
    scratch_types=dict(
        deg=pltpu.VMEM_SHARED((NZ,), jnp.float32),
        rowi=pltpu.VMEM((NCHUNK // 4, 1, 128), jnp.int32),
        rowu=pltpu.VMEM((NCHUNK // 2, 1, 128), jnp.int32),
        vbuf=pltpu.VMEM((128,), jnp.float32),
        sem=pltpu.SemaphoreType.DMA,
    ),
)
def _sc_deg(row_hbm, z640_hbm, out0_hbm, out1_hbm, deg, rowi, rowu, vbuf,
            sem):
    c = lax.axis_index("c")
    s = lax.axis_index("s")
    w = c * NS + s
    pltpu.sync_copy(row_hbm.at[w], rowi)
    pltpu.sync_copy(z640_hbm, deg.at[pl.ds(s * 640, 640)])

    def fill(k, _):
        vbuf[pl.ds(k * 16, 16)] = jnp.full((16,), 1.0, jnp.float32)
        return 0

    lax.fori_loop(0, 128 // 16, fill, 0)

    def unpack(j, _):
        jw = j // 4
        off = (j % 4) * (CH // 2)

        def up(g, _):
            vr = rowi[jw, 0, pl.ds(off + g * 16, 16)]
            rowu[j // 2, 0, pl.ds((j % 2) * CH + g * 32, 16)] = vr & 0xFFFF
            rowu[j // 2, 0, pl.ds((j % 2) * CH + g * 32 + 16, 16)] = (
                lax.shift_right_logical(vr, 16))
            return 0

        lax.fori_loop(0, CH // 32, up, 0)
        return 0

    lax.fori_loop(0, NCHUNK, unpack, 0)
    plsc.subcore_barrier()

    DEPTH = 8
    NSC = NCHUNK // 2
    for j in range(DEPTH):
        pltpu.async_copy(vbuf, deg.at[rowu.at[j, 0]], sem, add=True)

    def chunk(j, _):
        pltpu.make_async_copy(vbuf, deg.at[rowu.at[0, 0]], sem).wait()
        pltpu.async_copy(vbuf, deg.at[rowu.at[j, 0]], sem, add=True)
        return 0

    lax.fori_loop(DEPTH, NSC, chunk, 0)
    for j in range(DEPTH):
        pltpu.make_async_copy(vbuf, deg.at[rowu.at[0, 0]], sem).wait()
    plsc.subcore_barrier()

    @pl.when(c == 0)
    def _():
        pltpu.sync_copy(deg.at[pl.ds(s * 640, 640)], out0_hbm.at[pl.ds(s * 640, 640)])

    @pl.when(c == 1)
    def _():
        pltpu.sync_copy(deg.at[pl.ds(s * 640, 640)], out1_hbm.at[pl.ds(s * 640, 640)])


@functools.partial(
    pl.kernel,
    out_type=[jax.ShapeDtypeStruct((NS, 640, D), jnp.float32),
              jax.ShapeDtypeStruct((NS, 640, D), jnp.float32)],
    mesh=_mesh,
    compiler_params=pltpu.CompilerParams(needs_layout_passes=False),
    scratch_types=dict(
        agg=pltpu.VMEM_SHARED((NZ, D), jnp.float32),
        rowi=pltpu.VMEM((NCHUNK // 4, 1, 128), jnp.int32),
        coli=pltpu.VMEM((NCHUNK // 4, 1, 128), jnp.int32),
        rowb=pltpu.VMEM((3, 1, CH), jnp.int32),
        colb=pltpu.VMEM((3, 1, CH), jnp.int32),
        ab=pltpu.VMEM((3, 2, CH), jnp.float32),
        rows0=pltpu.VMEM((CH, D), jnp.float32),
        rows1=pltpu.VMEM((CH, D), jnp.float32),
        rows2=pltpu.VMEM((CH, D), jnp.float32),
        sab0=pltpu.SemaphoreType.DMA,
        sab1=pltpu.SemaphoreType.DMA,
        sab2=pltpu.SemaphoreType.DMA,
        sr0=pltpu.SemaphoreType.DMA,
        sr1=pltpu.SemaphoreType.DMA,
        sr2=pltpu.SemaphoreType.DMA,
        ss0=pltpu.SemaphoreType.DMA,
        ss1=pltpu.SemaphoreType.DMA,
        ss2=pltpu.SemaphoreType.DMA,
    ),
)
def _sc_layer(hh_hbm, a_hbm, b_hbm, row_hbm, col_hbm, zrows_hbm,
              out0_hbm, out1_hbm, agg, rowi, coli, rowb, colb, ab,
              rows0, rows1, rows2,
              sab0, sab1, sab2, sr0, sr1, sr2, ss0, ss1, ss2):
    c = lax.axis_index("c")
    s = lax.axis_index("s")
    w = c * NS + s
    pltpu.sync_copy(row_hbm.at[w], rowi)
    pltpu.sync_copy(col_hbm.at[w], coli)
    pltpu.sync_copy(zrows_hbm, agg.at[pl.ds(s * 640, 640)])
    plsc.subcore_barrier()

    rows_ = (rows0, rows1, rows2)
    sab_ = (sab0, sab1, sab2)
    sr_ = (sr0, sr1, sr2)
    ss_ = (ss0, ss1, ss2)

    def unpack_issue_gathers(j, k):
        jw = j // 4
        off = (j % 4) * (CH // 2)

        def up(g, _):
            vr = rowi[jw, 0, pl.ds(off + g * 16, 16)]
            vc = coli[jw, 0, pl.ds(off + g * 16, 16)]
            rowb[k, 0, pl.ds(g * 32, 16)] = vr & 0xFFFF
            rowb[k, 0, pl.ds(g * 32 + 16, 16)] = lax.shift_right_logical(vr, 16)
            colb[k, 0, pl.ds(g * 32, 16)] = vc & 0xFFFF
            colb[k, 0, pl.ds(g * 32 + 16, 16)] = lax.shift_right_logical(vc, 16)
            return 0

        lax.fori_loop(0, CH // 32, up, 0)
        pltpu.async_copy(a_hbm.at[rowb.at[k, 0]], ab.at[k, 0], sab_[k])
        pltpu.async_copy(b_hbm.at[colb.at[k, 0]], ab.at[k, 1], sab_[k])
        pltpu.async_copy(hh_hbm.at[rowb.at[k, 0]], rows_[k], sr_[k])

    def wait_gathers(k):
        pltpu.make_async_copy(a_hbm.at[rowb.at[k, 0]], ab.at[k, 0], sab_[k]).wait()
        pltpu.make_async_copy(b_hbm.at[colb.at[k, 0]], ab.at[k, 1], sab_[k]).wait()
        pltpu.make_async_copy(hh_hbm.at[rowb.at[k, 0]], rows_[k], sr_[k]).wait()

    def wait_scatter(k):
        pltpu.make_async_copy(rows_[k], agg.at[colb.at[k, 0]], ss_[k]).wait()

    def compute(j, k):
        def group(g, _):
            t = ab[k, 0, pl.ds(g * 16, 16)] + ab[k, 1, pl.ds(g * 16, 16)]
            e = jnp.exp(-2.0 * jnp.abs(t))
            th = (1.0 - e) / (1.0 + e)
            th = jnp.where(t < 0.0, -th, th)
            r = rows_[k]
            for l in range(16):
                ce = th[l]
                ei = g * 16 + l
                for q in range(D // 16):
                    r[ei, pl.ds(q * 16, 16)] = r[ei, pl.ds(q * 16, 16)] * ce
            return 0

        lax.fori_loop(0, CH // 16, group, 0)
        pltpu.async_copy(rows_[k], agg.at[colb.at[k, 0]], ss_[k], add=True)

    def step(j, k, kn, wait_sc, issue):
        if wait_sc:
            wait_scatter(kn)
        if issue:
            unpack_issue_gathers(j + 1, kn)
        wait_gathers(k)
        compute(j, k)

    unpack_issue_gathers(0, 0)
    step(0, 0, 1, False, True)
    step(1, 1, 2, False, True)
    step(2, 2, 0, True, True)

    def body(i, _):
        j = 3 * i
        step(j, 0, 1, True, True)
        step(j + 1, 1, 2, True, True)
        step(j + 2, 2, 0, True, True)
        return 0

    lax.fori_loop(1, NPROC // 3 - 1, body, 0)
    j = NPROC - 3
    step(j, 0, 1, True, True)
    step(j + 1, 1, 2, True, True)
    step(j + 2, 2, 0, True, False)
    wait_scatter(1)
    wait_scatter(2)

    plsc.subcore_barrier()

    @pl.when(c == 0)
    def _():
        pltpu.sync_copy(agg.at[pl.ds(s * 640, 640)], out0_hbm.at[s])

    @pl.when(c == 1)
    def _():
        pltpu.sync_copy(agg.at[pl.ds(s * 640, 640)], out1_hbm.at[s])


def _tc_a_body(x_ref, w1_ref, b1_ref, wg_ref, bg_ref, d0_ref, d1_ref,
               h_ref, hh_ref, a_ref, b_ref, nd_ref):
    nd = lax.rsqrt(jnp.maximum(d0_ref[...] + d1_ref[...], 1.0))
    nd_ref[...] = nd
    h = lax.dot_general(x_ref[...], w1_ref[...], (((1,), (1,)), ((), ())),
                        preferred_element_type=jnp.float32)
    h = jnp.maximum(h + b1_ref[...], 0.0)
    h_ref[...] = h
    hh_ref[:N_NODES, :] = h * nd[:N_NODES, :]
    g2 = lax.dot_general(wg_ref[...], h, (((1,), (1,)), ((), ())),
                         preferred_element_type=jnp.float32)
    a_ref[...] = g2[0:1, :] + bg_ref[...]
    b_ref[...] = g2[1:2, :]


def _tc_a(x, w1, b1r, wg2, bgr, d0, d1):
    return pl.pallas_call(
        _tc_a_body,
        out_shape=[
            jax.ShapeDtypeStruct((N_NODES, D), jnp.float32),
            jax.ShapeDtypeStruct((NP, D), jnp.float32),
            jax.ShapeDtypeStruct((1, N_NODES), jnp.float32),
            jax.ShapeDtypeStruct((1, N_NODES), jnp.float32),
            jax.ShapeDtypeStruct((NZ, 1), jnp.float32),
        ],
    )(x, w1, b1r, wg2, bgr, d0, d1)


def _tc_b_body(hraw_ref, agg0_ref, agg1_ref, nd_ref, wg_ref, bg_ref,
               h_ref, hh_ref, a_ref, b_ref):
    nd = nd_ref[:N_NODES, :]
    h = EPS * hraw_ref[...] + nd * (agg0_ref[:N_NODES, :]
                                    + agg1_ref[:N_NODES, :])
    h_ref[...] = h
    hh_ref[:N_NODES, :] = h * nd
    g2 = lax.dot_general(wg_ref[...], h, (((1,), (1,)), ((), ())),
                         preferred_element_type=jnp.float32)
    a_ref[...] = g2[0:1, :] + bg_ref[...]
    b_ref[...] = g2[1:2, :]


def _tc_b(hraw, agg0, agg1, ndc, wg2, bgr):
    return pl.pallas_call(
        _tc_b_body,
        out_shape=[
            jax.ShapeDtypeStruct((N_NODES, D), jnp.float32),
            jax.ShapeDtypeStruct((NP, D), jnp.float32),
            jax.ShapeDtypeStruct((1, N_NODES), jnp.float32),
            jax.ShapeDtypeStruct((1, N_NODES), jnp.float32),
        ],
    )(hraw, agg0, agg1, ndc, wg2, bgr)


def _tc_c_body(hraw_ref, agg0_ref, agg1_ref, nd_ref, w2_ref, b2_ref, out_ref):
    nd = nd_ref[:N_NODES, :]
    h = EPS * hraw_ref[...] + nd * (agg0_ref[:N_NODES, :]
                                    + agg1_ref[:N_NODES, :])
    logits = lax.dot_general(h, w2_ref[...], (((1,), (1,)), ((), ())),
                             preferred_element_type=jnp.float32)
    logits = logits + b2_ref[...]
    m = jnp.max(logits, axis=1, keepdims=True)
    z = logits - m
    lse = jnp.log(jnp.sum(jnp.exp(z), axis=1, keepdims=True))
    out_ref[...] = z - lse


def _tc_c(hraw, agg0, agg1, ndc, w2, b2r):
    ncls = w2.shape[0]
    return pl.pallas_call(
        _tc_c_body,
        out_shape=jax.ShapeDtypeStruct((N_NODES, ncls), jnp.float32),
    )(hraw, agg0, agg1, ndc, w2, b2r)


def kernel(x, edge_index, W1, b1, Wg0, bg0, Wg1, bg1, W2, b2):
    row = edge_index[0]
    col = edge_index[1]
    pad = EPT_PAD - EPT
    def pack_u16(v):
        v = jnp.pad(v.reshape(NW, EPT), ((0, 0), (0, pad)),
                    constant_values=N_NODES)
        v = v.reshape(NW, NCHUNK, CH // 32, 2, 16)
        packed = v[:, :, :, 0, :] | (v[:, :, :, 1, :] << 16)
        return packed.reshape(NW, NCHUNK // 4, 1, 128)

    rowp = pack_u16(row)
    colp = pack_u16(col)
    z640 = jnp.zeros((640,), jnp.float32)
    zrows = jnp.zeros((640, D), jnp.float32)

    deg0, deg1 = _sc_deg(rowp, z640)
    h, hh, ar, br, ndc = _tc_a(x, W1, b1.reshape(1, D), Wg0.reshape(2, D),
                               bg0.reshape(1, 1), deg0.reshape(NZ, 1),
                               deg1.reshape(NZ, 1))

    ap = jnp.pad(ar.reshape(N_NODES), (0, NP - N_NODES))
    bp = jnp.pad(br.reshape(N_NODES), (0, NP - N_NODES))
    a0, a1 = _sc_layer(hh, ap, bp, rowp, colp, zrows)

    h1, hh1, ar1, br1 = _tc_b(h, a0.reshape(NZ, D), a1.reshape(NZ, D), ndc,
                              Wg1.reshape(2, D), bg1.reshape(1, 1))
    a1p = jnp.pad(ar1.reshape(N_NODES), (0, NP - N_NODES))
    b1p = jnp.pad(br1.reshape(N_NODES), (0, NP - N_NODES))
    c0, c1 = _sc_layer(hh1, a1p, b1p, rowp, colp, zrows)

    return _tc_c(h, c0.reshape(NZ, D), c1.reshape(NZ, D), ndc, W2,
                 b2.reshape(1, 40))
